# speedup vs baseline: 17.2809x; 17.2809x over previous
import jax
import jax.numpy as jnp
from jax import lax
from jax.experimental import pallas as pl
from jax.experimental.pallas import tpu as pltpu
from jax.experimental.pallas import tpu_sc as plsc

N = 10000
E = 320000
D = 128
H = 128
G = 64

NC = 2
NS = 16
NW = NC * NS
EP = E // NW
K = 80
NITER = EP // K
N_PAD = 10240
RPS = N_PAD // NS
BLK = 2048
NBLK = N_PAD // BLK

_MESH = plsc.VectorSubcoreMesh(
    core_axis_name="c", subcore_axis_name="s", num_cores=NC, num_subcores=NS
)


def _sc_deg_body(dst_hbm, ones_hbm, zeros_hbm, out_hbm, deg_sh, idx_v, ones_v):
    c = lax.axis_index("c")
    s = lax.axis_index("s")
    wid = s * NC + c
    pltpu.sync_copy(zeros_hbm, deg_sh.at[pl.ds(s * RPS, RPS)])
    pltpu.sync_copy(ones_hbm, ones_v)
    plsc.subcore_barrier()

    def body(i, carry):
        base = wid * EP + i * K
        pltpu.sync_copy(dst_hbm.at[pl.ds(base, K)], idx_v)
        pltpu.sync_copy(ones_v, deg_sh.at[idx_v], add=True)
        return carry

    lax.fori_loop(0, NITER, body, 0)
    plsc.subcore_barrier()
    pltpu.sync_copy(deg_sh.at[pl.ds(s * RPS, RPS)], out_hbm.at[c, pl.ds(s * RPS, RPS)])


_sc_deg = pl.kernel(
    _sc_deg_body,
    out_type=jax.ShapeDtypeStruct((NC, N_PAD), jnp.float32),
    mesh=_MESH,
    scratch_types=[
        pltpu.VMEM_SHARED((N_PAD,), jnp.float32),
        pltpu.VMEM((K,), jnp.int32),
        pltpu.VMEM((K,), jnp.float32),
    ],
)


def _sc_scatter_body(hs_hbm, src_hbm, dst_hbm, zrows_hbm, out_hbm,
                     acc_sh, idx_s, idx_d, rows_v, sem):
    c = lax.axis_index("c")
    s = lax.axis_index("s")
    wid = s * NC + c
    pltpu.sync_copy(zrows_hbm, acc_sh.at[pl.ds(s * RPS, RPS)])
    plsc.subcore_barrier()

    def body(i, carry):
        base = wid * EP + i * K
        pltpu.sync_copy(src_hbm.at[pl.ds(base, K)], idx_s)
        pltpu.sync_copy(dst_hbm.at[pl.ds(base, K)], idx_d)
        pltpu.async_copy(hs_hbm.at[idx_s], rows_v, sem).wait()
        pltpu.sync_copy(rows_v, acc_sh.at[idx_d], add=True)
        return carry

    lax.fori_loop(0, NITER, body, 0)
    plsc.subcore_barrier()
    pltpu.sync_copy(acc_sh.at[pl.ds(s * RPS, RPS)],
                    out_hbm.at[c, pl.ds(s * RPS, RPS)])


_sc_scatter = pl.kernel(
    _sc_scatter_body,
    out_type=jax.ShapeDtypeStruct((NC, N_PAD, H), jnp.float32),
    mesh=_MESH,
    scratch_types=[
        pltpu.VMEM_SHARED((N_PAD, H), jnp.float32),
        pltpu.VMEM((K,), jnp.int32),
        pltpu.VMEM((K,), jnp.int32),
        pltpu.VMEM((K, H), jnp.float32),
        pltpu.SemaphoreType.DMA,
    ],
)


def _hs_body(x_ref, w_ref, b_ref, d0_ref, d1_ref, hs_ref, dinv_ref):
    deg = d0_ref[...] + d1_ref[...] + 1.0
    dinv = lax.rsqrt(deg)
    h = jnp.dot(x_ref[...], w_ref[...], preferred_element_type=jnp.float32)
    hs_ref[...] = (h + b_ref[...]) * dinv
    dinv_ref[...] = dinv


_hs_call = pl.pallas_call(
    _hs_body,
    grid=(NBLK,),
    in_specs=[
        pl.BlockSpec((BLK, D), lambda i: (i, 0)),
        pl.BlockSpec((D, H), lambda i: (0, 0)),
        pl.BlockSpec((1, H), lambda i: (0, 0)),
        pl.BlockSpec((BLK, 1), lambda i: (i, 0)),
        pl.BlockSpec((BLK, 1), lambda i: (i, 0)),
    ],
    out_specs=[
        pl.BlockSpec((BLK, H), lambda i: (i, 0)),
        pl.BlockSpec((BLK, 1), lambda i: (i, 0)),
    ],
    out_shape=[
        jax.ShapeDtypeStruct((N_PAD, H), jnp.float32),
        jax.ShapeDtypeStruct((N_PAD, 1), jnp.float32),
    ],
)


def _final_body(a0_ref, a1_ref, hs_ref, dinv_ref, bi_ref, wfc_ref, bfc_ref,
                out_ref, sums_ref, cnts_ref):
    i = pl.program_id(0)

    @pl.when(i == 0)
    def _():
        sums_ref[...] = jnp.zeros_like(sums_ref)
        cnts_ref[...] = jnp.zeros_like(cnts_ref)

    node = jnp.maximum(
        (a0_ref[...] + a1_ref[...] + hs_ref[...]) * dinv_ref[...], 0.0)
    onehot = (bi_ref[...] == lax.broadcasted_iota(jnp.int32, (1, G), 1))
    onehot = onehot.astype(jnp.float32)
    sums_ref[...] += lax.dot_general(
        onehot, node, (((0,), (0,)), ((), ())),
        preferred_element_type=jnp.float32)
    cnts_ref[...] += lax.dot_general(
        onehot, jnp.ones_like(node), (((0,), (0,)), ((), ())),
        preferred_element_type=jnp.float32)

    @pl.when(i == NBLK - 1)
    def _():
        pooled = sums_ref[...] / jnp.maximum(cnts_ref[...], 1.0)
        logits = jnp.dot(pooled, wfc_ref[...],
                         preferred_element_type=jnp.float32) + bfc_ref[...]
        m = jnp.max(logits, axis=-1, keepdims=True)
        lse = m + jnp.log(jnp.sum(jnp.exp(logits - m), axis=-1, keepdims=True))
        out_ref[...] = logits - lse


_final_call = pl.pallas_call(
    _final_body,
    grid=(NBLK,),
    in_specs=[
        pl.BlockSpec((BLK, H), lambda i: (i, 0)),
        pl.BlockSpec((BLK, H), lambda i: (i, 0)),
        pl.BlockSpec((BLK, H), lambda i: (i, 0)),
        pl.BlockSpec((BLK, 1), lambda i: (i, 0)),
        pl.BlockSpec((BLK, 1), lambda i: (i, 0)),
        pl.BlockSpec((H, 2), lambda i: (0, 0)),
        pl.BlockSpec((1, 2), lambda i: (0, 0)),
    ],
    out_specs=pl.BlockSpec((G, 2), lambda i: (0, 0)),
    out_shape=jax.ShapeDtypeStruct((G, 2), jnp.float32),
    scratch_shapes=[
        pltpu.VMEM((G, H), jnp.float32),
        pltpu.VMEM((G, H), jnp.float32),
    ],
)


def kernel(x, edge_index, batch_idx, W1, b1, Wfc, bfc):
    src = edge_index[0]
    dst = edge_index[1]
    x_pad = jnp.pad(x, ((0, N_PAD - N), (0, 0)))
    bi_pad = jnp.pad(batch_idx, (0, N_PAD - N),
                     constant_values=G).reshape(N_PAD, 1)
    ones_k = jnp.ones((K,), jnp.float32)
    zeros_r = jnp.zeros((RPS,), jnp.float32)
    zrows = jnp.zeros((RPS, H), jnp.float32)

    deg_parts = _sc_deg(dst, ones_k, zeros_r)
    d0 = deg_parts[0].reshape(N_PAD, 1)
    d1 = deg_parts[1].reshape(N_PAD, 1)
    hs, dinv = _hs_call(x_pad, W1, b1.reshape(1, H), d0, d1)
    acc = _sc_scatter(hs, src, dst, zrows)
    return _final_call(acc[0], acc[1], hs, dinv, bi_pad, Wfc,
                       bfc.reshape(1, 2))


# speedup vs baseline: 36.5612x; 2.1157x over previous
import jax
import jax.numpy as jnp
from jax import lax
from jax.experimental import pallas as pl
from jax.experimental.pallas import tpu as pltpu
from jax.experimental.pallas import tpu_sc as plsc

N = 10000
E = 320000
D = 128
H = 128
G = 64

NC = 2
NS = 16
NW = NC * NS
EP = E // NW
K = 40
NITER = EP // K
N_PAD = 10240
RPS = N_PAD // NS
BLK = 2048
NBLK = N_PAD // BLK

_MESH = plsc.VectorSubcoreMesh(
    core_axis_name="c", subcore_axis_name="s", num_cores=NC, num_subcores=NS
)


W_DEG = 8
NB = 5
IB = 25
NSB = NITER // IB


def _sc_deg_body(dst_hbm, ones_hbm, zeros_hbm, out_hbm, deg_sh, idx_d2, ones_v,
                 semd):
    c = lax.axis_index("c")
    s = lax.axis_index("s")
    wid = s * NC + c
    pltpu.sync_copy(zeros_hbm, deg_sh.at[pl.ds(s * RPS, RPS)])
    pltpu.sync_copy(ones_hbm, ones_v)
    plsc.subcore_barrier()

    def superblock(sb, carry):
        pltpu.sync_copy(dst_hbm.at[wid, sb], idx_d2)

        def body(j, c2):
            pltpu.async_copy(ones_v, deg_sh.at[idx_d2.at[j]], semd, add=True)

            @pl.when(j >= W_DEG)
            def _():
                pltpu.make_async_copy(ones_v, deg_sh.at[idx_d2.at[0]],
                                      semd).wait()

            return c2

        lax.fori_loop(0, IB, body, 0)

        def drain(j, c2):
            pltpu.make_async_copy(ones_v, deg_sh.at[idx_d2.at[0]], semd).wait()
            return c2

        lax.fori_loop(0, W_DEG, drain, 0)
        return carry

    lax.fori_loop(0, NSB, superblock, 0)
    plsc.subcore_barrier()
    pltpu.sync_copy(deg_sh.at[pl.ds(s * RPS, RPS)], out_hbm.at[c, pl.ds(s * RPS, RPS)])


_sc_deg = pl.kernel(
    _sc_deg_body,
    out_type=jax.ShapeDtypeStruct((NC, N_PAD), jnp.float32),
    mesh=_MESH,
    scratch_types=[
        pltpu.VMEM_SHARED((N_PAD,), jnp.float32),
        pltpu.VMEM((IB, K), jnp.int32),
        pltpu.VMEM((K,), jnp.float32),
        pltpu.SemaphoreType.DMA,
    ],
)


def _sc_scatter_body(hs_hbm, src_hbm, dst_hbm, zrows_hbm, out_hbm,
                     acc_sh, idx_s2, idx_d2, rows_v,
                     semg0, semg1, semg2, semg3, semg4,
                     sems0, sems1, sems2, sems3, sems4):
    semg = (semg0, semg1, semg2, semg3, semg4)
    sems = (sems0, sems1, sems2, sems3, sems4)
    c = lax.axis_index("c")
    s = lax.axis_index("s")
    wid = s * NC + c
    pltpu.sync_copy(zrows_hbm, acc_sh.at[pl.ds(s * RPS, RPS)])
    plsc.subcore_barrier()

    def superblock(sb, carry):
        pltpu.sync_copy(src_hbm.at[wid, sb], idx_s2)
        pltpu.sync_copy(dst_hbm.at[wid, sb], idx_d2)
        for b in range(NB):
            pltpu.async_copy(hs_hbm.at[idx_s2.at[b]], rows_v.at[b], semg[b])

        def inner(o, c2):
            for b in range(NB):
                j = o * NB + b
                pltpu.make_async_copy(hs_hbm.at[idx_s2.at[b]], rows_v.at[b],
                                      semg[b]).wait()
                sd = pltpu.async_copy(rows_v.at[b], acc_sh.at[idx_d2.at[j]],
                                      sems[b], add=True)
                sd.wait()

                @pl.when(j + NB < IB)
                def _():
                    pltpu.async_copy(hs_hbm.at[idx_s2.at[j + NB]],
                                     rows_v.at[b], semg[b])

            return c2

        lax.fori_loop(0, IB // NB, inner, 0)
        return carry

    lax.fori_loop(0, NSB, superblock, 0)
    plsc.subcore_barrier()
    pltpu.sync_copy(acc_sh.at[pl.ds(s * RPS, RPS)],
                    out_hbm.at[c, pl.ds(s * RPS, RPS)])


_sc_scatter = pl.kernel(
    _sc_scatter_body,
    out_type=jax.ShapeDtypeStruct((NC, N_PAD, H), jnp.float32),
    mesh=_MESH,
    scratch_types=[
        pltpu.VMEM_SHARED((N_PAD, H), jnp.float32),
        pltpu.VMEM((IB, K), jnp.int32),
        pltpu.VMEM((IB, K), jnp.int32),
        pltpu.VMEM((NB, K, H), jnp.float32),
        pltpu.SemaphoreType.DMA,
        pltpu.SemaphoreType.DMA,
        pltpu.SemaphoreType.DMA,
        pltpu.SemaphoreType.DMA,
        pltpu.SemaphoreType.DMA,
        pltpu.SemaphoreType.DMA,
        pltpu.SemaphoreType.DMA,
        pltpu.SemaphoreType.DMA,
        pltpu.SemaphoreType.DMA,
        pltpu.SemaphoreType.DMA,
    ],
)


def _hs_body(x_ref, w_ref, b_ref, d0_ref, d1_ref, hs_ref, dinv_ref):
    deg = d0_ref[...] + d1_ref[...] + 1.0
    dinv = lax.rsqrt(deg)
    h = jnp.dot(x_ref[...], w_ref[...], preferred_element_type=jnp.float32)
    hs_ref[...] = (h + b_ref[...]) * dinv
    dinv_ref[...] = dinv


_hs_call = pl.pallas_call(
    _hs_body,
    grid=(NBLK,),
    in_specs=[
        pl.BlockSpec((BLK, D), lambda i: (i, 0)),
        pl.BlockSpec((D, H), lambda i: (0, 0)),
        pl.BlockSpec((1, H), lambda i: (0, 0)),
        pl.BlockSpec((BLK, 1), lambda i: (i, 0)),
        pl.BlockSpec((BLK, 1), lambda i: (i, 0)),
    ],
    out_specs=[
        pl.BlockSpec((BLK, H), lambda i: (i, 0)),
        pl.BlockSpec((BLK, 1), lambda i: (i, 0)),
    ],
    out_shape=[
        jax.ShapeDtypeStruct((N_PAD, H), jnp.float32),
        jax.ShapeDtypeStruct((N_PAD, 1), jnp.float32),
    ],
)


def _final_body(a0_ref, a1_ref, hs_ref, dinv_ref, bi_ref, wfc_ref, bfc_ref,
                out_ref, sums_ref, cnts_ref):
    i = pl.program_id(0)

    @pl.when(i == 0)
    def _():
        sums_ref[...] = jnp.zeros_like(sums_ref)
        cnts_ref[...] = jnp.zeros_like(cnts_ref)

    node = jnp.maximum(
        (a0_ref[...] + a1_ref[...] + hs_ref[...]) * dinv_ref[...], 0.0)
    onehot = (bi_ref[...] == lax.broadcasted_iota(jnp.int32, (1, G), 1))
    onehot = onehot.astype(jnp.float32)
    sums_ref[...] += lax.dot_general(
        onehot, node, (((0,), (0,)), ((), ())),
        preferred_element_type=jnp.float32)
    cnts_ref[...] += lax.dot_general(
        onehot, jnp.ones_like(node), (((0,), (0,)), ((), ())),
        preferred_element_type=jnp.float32)

    @pl.when(i == NBLK - 1)
    def _():
        pooled = sums_ref[...] / jnp.maximum(cnts_ref[...], 1.0)
        logits = jnp.dot(pooled, wfc_ref[...],
                         preferred_element_type=jnp.float32) + bfc_ref[...]
        m = jnp.max(logits, axis=-1, keepdims=True)
        lse = m + jnp.log(jnp.sum(jnp.exp(logits - m), axis=-1, keepdims=True))
        out_ref[...] = logits - lse


_final_call = pl.pallas_call(
    _final_body,
    grid=(NBLK,),
    in_specs=[
        pl.BlockSpec((BLK, H), lambda i: (i, 0)),
        pl.BlockSpec((BLK, H), lambda i: (i, 0)),
        pl.BlockSpec((BLK, H), lambda i: (i, 0)),
        pl.BlockSpec((BLK, 1), lambda i: (i, 0)),
        pl.BlockSpec((BLK, 1), lambda i: (i, 0)),
        pl.BlockSpec((H, 2), lambda i: (0, 0)),
        pl.BlockSpec((1, 2), lambda i: (0, 0)),
    ],
    out_specs=pl.BlockSpec((G, 2), lambda i: (0, 0)),
    out_shape=jax.ShapeDtypeStruct((G, 2), jnp.float32),
    scratch_shapes=[
        pltpu.VMEM((G, H), jnp.float32),
        pltpu.VMEM((G, H), jnp.float32),
    ],
)


def kernel(x, edge_index, batch_idx, W1, b1, Wfc, bfc):
    src = edge_index[0].reshape(NW, NSB, IB, K)
    dst = edge_index[1].reshape(NW, NSB, IB, K)
    x_pad = jnp.pad(x, ((0, N_PAD - N), (0, 0)))
    bi_pad = jnp.pad(batch_idx, (0, N_PAD - N),
                     constant_values=G).reshape(N_PAD, 1)
    ones_k = jnp.ones((K,), jnp.float32)
    zeros_r = jnp.zeros((RPS,), jnp.float32)
    zrows = jnp.zeros((RPS, H), jnp.float32)

    deg_parts = _sc_deg(dst, ones_k, zeros_r)
    d0 = deg_parts[0].reshape(N_PAD, 1)
    d1 = deg_parts[1].reshape(N_PAD, 1)
    hs, dinv = _hs_call(x_pad, W1, b1.reshape(1, H), d0, d1)
    acc = _sc_scatter(hs, src, dst, zrows)
    return _final_call(acc[0], acc[1], hs, dinv, bi_pad, Wfc,
                       bfc.reshape(1, 2))


# speedup vs baseline: 37.6919x; 1.0309x over previous
import jax
import jax.numpy as jnp
from jax import lax
from jax.experimental import pallas as pl
from jax.experimental.pallas import tpu as pltpu
from jax.experimental.pallas import tpu_sc as plsc

N = 10000
E = 320000
D = 128
H = 128
G = 64

NC = 2
NS = 16
NW = NC * NS
EP = E // NW
K = 40
NITER = EP // K
N_PAD = 10240
RPS = N_PAD // NS
BLK = 2048
NBLK = N_PAD // BLK

_MESH = plsc.VectorSubcoreMesh(
    core_axis_name="c", subcore_axis_name="s", num_cores=NC, num_subcores=NS
)


W_DEG = 8
NB = 5
F_G = 3
Q_S = 2
IB = 50
NSB = NITER // IB


def _sc_deg_body(dst_hbm, ones_hbm, zeros_hbm, out_hbm, deg_sh, idx_d2, ones_v,
                 semd):
    c = lax.axis_index("c")
    s = lax.axis_index("s")
    wid = s * NC + c
    pltpu.sync_copy(zeros_hbm, deg_sh.at[pl.ds(s * RPS, RPS)])
    pltpu.sync_copy(ones_hbm, ones_v)
    plsc.subcore_barrier()

    def superblock(sb, carry):
        pltpu.sync_copy(dst_hbm.at[wid, sb], idx_d2)

        def body(j, c2):
            pltpu.async_copy(ones_v, deg_sh.at[idx_d2.at[j]], semd, add=True)

            @pl.when(j >= W_DEG)
            def _():
                pltpu.make_async_copy(ones_v, deg_sh.at[idx_d2.at[0]],
                                      semd).wait()

            return c2

        lax.fori_loop(0, IB, body, 0)

        def drain(j, c2):
            pltpu.make_async_copy(ones_v, deg_sh.at[idx_d2.at[0]], semd).wait()
            return c2

        lax.fori_loop(0, W_DEG, drain, 0)
        return carry

    lax.fori_loop(0, NSB, superblock, 0)
    plsc.subcore_barrier()
    pltpu.sync_copy(deg_sh.at[pl.ds(s * RPS, RPS)], out_hbm.at[c, pl.ds(s * RPS, RPS)])


_sc_deg = pl.kernel(
    _sc_deg_body,
    out_type=jax.ShapeDtypeStruct((NC, N_PAD), jnp.float32),
    mesh=_MESH,
    scratch_types=[
        pltpu.VMEM_SHARED((N_PAD,), jnp.float32),
        pltpu.VMEM((IB, K), jnp.int32),
        pltpu.VMEM((K,), jnp.float32),
        pltpu.SemaphoreType.DMA,
    ],
)


def _sc_scatter_body(hs_hbm, src_hbm, dst_hbm, zrows_hbm, out_hbm,
                     acc_sh, idx_s2, idx_d2, rows_v,
                     semg0, semg1, semg2, semg3, semg4,
                     sems0, sems1, sems2, sems3, sems4):
    semg = (semg0, semg1, semg2, semg3, semg4)
    sems = (sems0, sems1, sems2, sems3, sems4)
    c = lax.axis_index("c")
    s = lax.axis_index("s")
    wid = s * NC + c
    @pl.when(c == 0)
    def _():
        pltpu.sync_copy(hs_hbm.at[pl.ds(s * RPS, RPS)],
                        acc_sh.at[pl.ds(s * RPS, RPS)])

    @pl.when(c != 0)
    def _():
        pltpu.sync_copy(zrows_hbm, acc_sh.at[pl.ds(s * RPS, RPS)])

    plsc.subcore_barrier()

    def superblock(sb, carry):
        pltpu.sync_copy(src_hbm.at[wid, sb], idx_s2)
        pltpu.sync_copy(dst_hbm.at[wid, sb], idx_d2)
        for b in range(F_G):
            pltpu.async_copy(hs_hbm.at[idx_s2.at[b]], rows_v.at[b], semg[b])

        def inner(o, c2):
            for b in range(NB):
                j = o * NB + b
                pltpu.make_async_copy(hs_hbm.at[idx_s2.at[b]], rows_v.at[b],
                                      semg[b]).wait()
                pltpu.async_copy(rows_v.at[b], acc_sh.at[idx_d2.at[j]],
                                 sems[b], add=True)
                bq = (b + NB - Q_S) % NB

                @pl.when(j >= Q_S)
                def _():
                    pltpu.make_async_copy(rows_v.at[bq],
                                          acc_sh.at[idx_d2.at[j]],
                                          sems[bq]).wait()

                @pl.when(j + F_G < IB)
                def _():
                    pltpu.async_copy(hs_hbm.at[idx_s2.at[j + F_G]],
                                     rows_v.at[bq], semg[bq])

            return c2

        lax.fori_loop(0, IB // NB, inner, 0)
        for j in range(IB - Q_S, IB):
            pltpu.make_async_copy(rows_v.at[j % NB],
                                  acc_sh.at[idx_d2.at[0]],
                                  sems[j % NB]).wait()
        return carry

    lax.fori_loop(0, NSB, superblock, 0)
    plsc.subcore_barrier()
    pltpu.sync_copy(acc_sh.at[pl.ds(s * RPS, RPS)],
                    out_hbm.at[c, pl.ds(s * RPS, RPS)])


_sc_scatter = pl.kernel(
    _sc_scatter_body,
    out_type=jax.ShapeDtypeStruct((NC, N_PAD, H), jnp.float32),
    mesh=_MESH,
    scratch_types=[
        pltpu.VMEM_SHARED((N_PAD, H), jnp.float32),
        pltpu.VMEM((IB, K), jnp.int32),
        pltpu.VMEM((IB, K), jnp.int32),
        pltpu.VMEM((NB, K, H), jnp.float32),
        pltpu.SemaphoreType.DMA,
        pltpu.SemaphoreType.DMA,
        pltpu.SemaphoreType.DMA,
        pltpu.SemaphoreType.DMA,
        pltpu.SemaphoreType.DMA,
        pltpu.SemaphoreType.DMA,
        pltpu.SemaphoreType.DMA,
        pltpu.SemaphoreType.DMA,
        pltpu.SemaphoreType.DMA,
        pltpu.SemaphoreType.DMA,
    ],
)


def _hs_body(x_ref, w_ref, b_ref, d0_ref, d1_ref, hs_ref, dinv_ref):
    deg = d0_ref[...] + d1_ref[...] + 1.0
    dinv = lax.rsqrt(deg)
    h = jnp.dot(x_ref[...], w_ref[...], preferred_element_type=jnp.float32)
    hs_ref[...] = (h + b_ref[...]) * dinv
    dinv_ref[...] = dinv


_hs_call = pl.pallas_call(
    _hs_body,
    grid=(NBLK,),
    in_specs=[
        pl.BlockSpec((BLK, D), lambda i: (i, 0)),
        pl.BlockSpec((D, H), lambda i: (0, 0)),
        pl.BlockSpec((1, H), lambda i: (0, 0)),
        pl.BlockSpec((BLK, 1), lambda i: (i, 0)),
        pl.BlockSpec((BLK, 1), lambda i: (i, 0)),
    ],
    out_specs=[
        pl.BlockSpec((BLK, H), lambda i: (i, 0)),
        pl.BlockSpec((BLK, 1), lambda i: (i, 0)),
    ],
    out_shape=[
        jax.ShapeDtypeStruct((N_PAD, H), jnp.float32),
        jax.ShapeDtypeStruct((N_PAD, 1), jnp.float32),
    ],
)


def _final_body(a0_ref, a1_ref, dinv_ref, bi_ref, wfc_ref, bfc_ref,
                out_ref, sums_ref, cnts_ref):
    i = pl.program_id(0)

    @pl.when(i == 0)
    def _():
        sums_ref[...] = jnp.zeros_like(sums_ref)
        cnts_ref[...] = jnp.zeros_like(cnts_ref)

    node = jnp.maximum((a0_ref[...] + a1_ref[...]) * dinv_ref[...], 0.0)
    onehot = (bi_ref[...] == lax.broadcasted_iota(jnp.int32, (1, G), 1))
    onehot = onehot.astype(jnp.float32)
    sums_ref[...] += lax.dot_general(
        onehot, node, (((0,), (0,)), ((), ())),
        preferred_element_type=jnp.float32)
    cnts_ref[...] += lax.dot_general(
        onehot, jnp.ones_like(node), (((0,), (0,)), ((), ())),
        preferred_element_type=jnp.float32)

    @pl.when(i == NBLK - 1)
    def _():
        pooled = sums_ref[...] / jnp.maximum(cnts_ref[...], 1.0)
        logits = jnp.dot(pooled, wfc_ref[...],
                         preferred_element_type=jnp.float32) + bfc_ref[...]
        m = jnp.max(logits, axis=-1, keepdims=True)
        lse = m + jnp.log(jnp.sum(jnp.exp(logits - m), axis=-1, keepdims=True))
        out_ref[...] = logits - lse


_final_call = pl.pallas_call(
    _final_body,
    grid=(NBLK,),
    in_specs=[
        pl.BlockSpec((BLK, H), lambda i: (i, 0)),
        pl.BlockSpec((BLK, H), lambda i: (i, 0)),
        pl.BlockSpec((BLK, 1), lambda i: (i, 0)),
        pl.BlockSpec((BLK, 1), lambda i: (i, 0)),
        pl.BlockSpec((H, 2), lambda i: (0, 0)),
        pl.BlockSpec((1, 2), lambda i: (0, 0)),
    ],
    out_specs=pl.BlockSpec((G, 2), lambda i: (0, 0)),
    out_shape=jax.ShapeDtypeStruct((G, 2), jnp.float32),
    scratch_shapes=[
        pltpu.VMEM((G, H), jnp.float32),
        pltpu.VMEM((G, H), jnp.float32),
    ],
)


def kernel(x, edge_index, batch_idx, W1, b1, Wfc, bfc):
    src = edge_index[0].reshape(NW, NSB, IB, K)
    dst = edge_index[1].reshape(NW, NSB, IB, K)
    x_pad = jnp.pad(x, ((0, N_PAD - N), (0, 0)))
    bi_pad = jnp.pad(batch_idx, (0, N_PAD - N),
                     constant_values=G).reshape(N_PAD, 1)
    ones_k = jnp.ones((K,), jnp.float32)
    zeros_r = jnp.zeros((RPS,), jnp.float32)
    zrows = jnp.zeros((RPS, H), jnp.float32)

    deg_parts = _sc_deg(dst, ones_k, zeros_r)
    d0 = deg_parts[0].reshape(N_PAD, 1)
    d1 = deg_parts[1].reshape(N_PAD, 1)
    hs, dinv = _hs_call(x_pad, W1, b1.reshape(1, H), d0, d1)
    acc = _sc_scatter(hs, src, dst, zrows)
    return _final_call(acc[0], acc[1], dinv, bi_pad, Wfc,
                       bfc.reshape(1, 2))


# speedup vs baseline: 39.0687x; 1.0365x over previous
import jax
import jax.numpy as jnp
from jax import lax
from jax.experimental import pallas as pl
from jax.experimental.pallas import tpu as pltpu
from jax.experimental.pallas import tpu_sc as plsc

N = 10000
E = 320000
D = 128
H = 128
G = 64

NC = 2
NS = 16
NW = NC * NS
EP = E // NW
N_PAD = 10240
RPS = N_PAD // NS
BLK = 2048
NBLK = N_PAD // BLK

W_DEG = 8
KD = 100
IBD = 20
NSBD = EP // (IBD * KD)

KS = 64
IB = 40
NSB = 4
ET_PAD = NSB * IB * KS
E_PAD = NW * ET_PAD
NPADR = N_PAD - N
NB = 4
F_G = 3
Q_S = 1

_MESH = plsc.VectorSubcoreMesh(
    core_axis_name="c", subcore_axis_name="s", num_cores=NC, num_subcores=NS
)


def _sc_deg_body(dst_hbm, ones_hbm, zeros_hbm, out_hbm, deg_sh, idx_d2, ones_v,
                 semd):
    c = lax.axis_index("c")
    s = lax.axis_index("s")
    wid = s * NC + c
    pltpu.sync_copy(zeros_hbm, deg_sh.at[pl.ds(s * RPS, RPS)])
    pltpu.sync_copy(ones_hbm, ones_v)
    plsc.subcore_barrier()

    def superblock(sb, carry):
        pltpu.sync_copy(dst_hbm.at[wid, sb], idx_d2)

        def body(j, c2):
            pltpu.async_copy(ones_v, deg_sh.at[idx_d2.at[j]], semd, add=True)

            @pl.when(j >= W_DEG)
            def _():
                pltpu.make_async_copy(ones_v, deg_sh.at[idx_d2.at[0]],
                                      semd).wait()

            return c2

        lax.fori_loop(0, IBD, body, 0)

        def drain(j, c2):
            pltpu.make_async_copy(ones_v, deg_sh.at[idx_d2.at[0]], semd).wait()
            return c2

        lax.fori_loop(0, W_DEG, drain, 0)
        return carry

    lax.fori_loop(0, NSBD, superblock, 0)
    plsc.subcore_barrier()
    pltpu.sync_copy(deg_sh.at[pl.ds(s * RPS, RPS)],
                    out_hbm.at[c, pl.ds(s * RPS, RPS)])


_sc_deg = pl.kernel(
    _sc_deg_body,
    out_type=jax.ShapeDtypeStruct((NC, N_PAD), jnp.float32),
    mesh=_MESH,
    scratch_types=[
        pltpu.VMEM_SHARED((N_PAD,), jnp.float32),
        pltpu.VMEM((IBD, KD), jnp.int32),
        pltpu.VMEM((KD,), jnp.float32),
        pltpu.SemaphoreType.DMA,
    ],
)


def _sc_scatter_body(hs_hbm, src_hbm, dst_hbm, zrows_hbm, out_hbm,
                     acc_sh, idx_s2, idx_d2, rows_v,
                     semg0, semg1, semg2, semg3,
                     sems0, sems1, sems2, sems3):
    semg = (semg0, semg1, semg2, semg3)
    sems = (sems0, sems1, sems2, sems3)
    c = lax.axis_index("c")
    s = lax.axis_index("s")
    wid = s * NC + c
    @pl.when(c == 0)
    def _():
        pltpu.sync_copy(hs_hbm.at[pl.ds(s * RPS, RPS)],
                        acc_sh.at[pl.ds(s * RPS, RPS)])

    @pl.when(c != 0)
    def _():
        pltpu.sync_copy(zrows_hbm, acc_sh.at[pl.ds(s * RPS, RPS)])

    plsc.subcore_barrier()

    def superblock(sb, carry):
        pltpu.sync_copy(src_hbm.at[wid, sb], idx_s2)
        pltpu.sync_copy(dst_hbm.at[wid, sb], idx_d2)
        for b in range(F_G):
            pltpu.async_copy(hs_hbm.at[idx_s2.at[b]], rows_v.at[b], semg[b])

        def inner(o, c2):
            for b in range(NB):
                j = o * NB + b
                pltpu.make_async_copy(hs_hbm.at[idx_s2.at[b]], rows_v.at[b],
                                      semg[b]).wait()
                pltpu.async_copy(rows_v.at[b], acc_sh.at[idx_d2.at[j]],
                                 sems[b], add=True)
                bq = (b + NB - Q_S) % NB

                @pl.when(j >= Q_S)
                def _():
                    pltpu.make_async_copy(rows_v.at[bq],
                                          acc_sh.at[idx_d2.at[j]],
                                          sems[bq]).wait()

                @pl.when(j + F_G < IB)
                def _():
                    pltpu.async_copy(hs_hbm.at[idx_s2.at[j + F_G]],
                                     rows_v.at[bq], semg[bq])

            return c2

        lax.fori_loop(0, IB // NB, inner, 0)
        for j in range(IB - Q_S, IB):
            pltpu.make_async_copy(rows_v.at[j % NB], acc_sh.at[idx_d2.at[0]],
                                  sems[j % NB]).wait()
        return carry

    lax.fori_loop(0, NSB, superblock, 0)
    plsc.subcore_barrier()
    pltpu.sync_copy(acc_sh.at[pl.ds(s * RPS, RPS)],
                    out_hbm.at[c, pl.ds(s * RPS, RPS)])


_sc_scatter = pl.kernel(
    _sc_scatter_body,
    out_type=jax.ShapeDtypeStruct((NC, N_PAD, H), jnp.float32),
    mesh=_MESH,
    scratch_types=[
        pltpu.VMEM_SHARED((N_PAD, H), jnp.float32),
        pltpu.VMEM((IB, KS), jnp.int32),
        pltpu.VMEM((IB, KS), jnp.int32),
        pltpu.VMEM((NB, KS, H), jnp.float32),
    ] + [pltpu.SemaphoreType.DMA] * (2 * NB),
)


def _hs_body(x_ref, w_ref, b_ref, d0_ref, d1_ref, hs_ref, dinv_ref):
    deg = d0_ref[...] + d1_ref[...] + 1.0
    dinv = lax.rsqrt(deg)
    h = jnp.dot(x_ref[...], w_ref[...], preferred_element_type=jnp.float32)
    hs_ref[...] = (h + b_ref[...]) * dinv
    dinv_ref[...] = dinv


_hs_call = pl.pallas_call(
    _hs_body,
    grid=(NBLK,),
    in_specs=[
        pl.BlockSpec((BLK, D), lambda i: (i, 0)),
        pl.BlockSpec((D, H), lambda i: (0, 0)),
        pl.BlockSpec((1, H), lambda i: (0, 0)),
        pl.BlockSpec((BLK, 1), lambda i: (i, 0)),
        pl.BlockSpec((BLK, 1), lambda i: (i, 0)),
    ],
    out_specs=[
        pl.BlockSpec((BLK, H), lambda i: (i, 0)),
        pl.BlockSpec((BLK, 1), lambda i: (i, 0)),
    ],
    out_shape=[
        jax.ShapeDtypeStruct((N_PAD, H), jnp.float32),
        jax.ShapeDtypeStruct((N_PAD, 1), jnp.float32),
    ],
)


def _final_body(a0_ref, a1_ref, dinv_ref, bi_ref, wfc_ref, bfc_ref,
                out_ref, sums_ref, cnts_ref):
    i = pl.program_id(0)

    @pl.when(i == 0)
    def _():
        sums_ref[...] = jnp.zeros_like(sums_ref)
        cnts_ref[...] = jnp.zeros_like(cnts_ref)

    node = jnp.maximum((a0_ref[...] + a1_ref[...]) * dinv_ref[...], 0.0)
    onehot = (bi_ref[...] == lax.broadcasted_iota(jnp.int32, (1, G), 1))
    onehot = onehot.astype(jnp.float32)
    sums_ref[...] += lax.dot_general(
        onehot, node, (((0,), (0,)), ((), ())),
        preferred_element_type=jnp.float32)
    cnts_ref[...] += lax.dot_general(
        onehot, jnp.ones_like(node), (((0,), (0,)), ((), ())),
        preferred_element_type=jnp.float32)

    @pl.when(i == NBLK - 1)
    def _():
        pooled = sums_ref[...] / jnp.maximum(cnts_ref[...], 1.0)
        logits = jnp.dot(pooled, wfc_ref[...],
                         preferred_element_type=jnp.float32) + bfc_ref[...]
        m = jnp.max(logits, axis=-1, keepdims=True)
        lse = m + jnp.log(jnp.sum(jnp.exp(logits - m), axis=-1, keepdims=True))
        out_ref[...] = logits - lse


_final_call = pl.pallas_call(
    _final_body,
    grid=(NBLK,),
    in_specs=[
        pl.BlockSpec((BLK, H), lambda i: (i, 0)),
        pl.BlockSpec((BLK, H), lambda i: (i, 0)),
        pl.BlockSpec((BLK, 1), lambda i: (i, 0)),
        pl.BlockSpec((BLK, 1), lambda i: (i, 0)),
        pl.BlockSpec((H, 2), lambda i: (0, 0)),
        pl.BlockSpec((1, 2), lambda i: (0, 0)),
    ],
    out_specs=pl.BlockSpec((G, 2), lambda i: (0, 0)),
    out_shape=jax.ShapeDtypeStruct((G, 2), jnp.float32),
    scratch_shapes=[
        pltpu.VMEM((G, H), jnp.float32),
        pltpu.VMEM((G, H), jnp.float32),
    ],
)


def kernel(x, edge_index, batch_idx, W1, b1, Wfc, bfc):
    dst_deg = edge_index[1].reshape(NW, NSBD, IBD, KD)
    pad_rows = N + jnp.arange(E_PAD - E, dtype=jnp.int32) % NPADR
    src_t = jnp.concatenate([edge_index[0], pad_rows]).reshape(
        NW, NSB, IB, KS)
    dst_t = jnp.concatenate([edge_index[1], pad_rows]).reshape(
        NW, NSB, IB, KS)
    x_pad = jnp.pad(x, ((0, N_PAD - N), (0, 0)))
    bi_pad = jnp.pad(batch_idx, (0, N_PAD - N),
                     constant_values=G).reshape(N_PAD, 1)
    ones_k = jnp.ones((KD,), jnp.float32)
    zeros_r = jnp.zeros((RPS,), jnp.float32)
    zrows = jnp.zeros((RPS, H), jnp.float32)

    deg_parts = _sc_deg(dst_deg, ones_k, zeros_r)
    d0 = deg_parts[0].reshape(N_PAD, 1)
    d1 = deg_parts[1].reshape(N_PAD, 1)
    hs, dinv = _hs_call(x_pad, W1, b1.reshape(1, H), d0, d1)
    acc = _sc_scatter(hs, src_t, dst_t, zrows)
    return _final_call(acc[0], acc[1], dinv, bi_pad, Wfc,
                       bfc.reshape(1, 2))


# speedup vs baseline: 39.2771x; 1.0053x over previous
import jax
import jax.numpy as jnp
from jax import lax
from jax.experimental import pallas as pl
from jax.experimental.pallas import tpu as pltpu
from jax.experimental.pallas import tpu_sc as plsc

N = 10000
E = 320000
D = 128
H = 128
G = 64

NC = 2
NS = 16
NW = NC * NS
EP = E // NW
N_PAD = 10240
RPS = N_PAD // NS
BLK = 2000
NBLK = N // BLK

W_DEG = 8
KD = 100
IBD = 20
NSBD = EP // (IBD * KD)

KS = 64
IB = 40
NSB = 4
ET_PAD = NSB * IB * KS
E_PAD = NW * ET_PAD
NPADR = N_PAD - N
NB = 4
F_G = 3
Q_S = 1

_MESH = plsc.VectorSubcoreMesh(
    core_axis_name="c", subcore_axis_name="s", num_cores=NC, num_subcores=NS
)


def _sc_deg_body(dst_hbm, ones_hbm, zeros_hbm, out_hbm, deg_sh, idx_d2, ones_v,
                 semd):
    c = lax.axis_index("c")
    s = lax.axis_index("s")
    wid = s * NC + c
    pltpu.sync_copy(zeros_hbm, deg_sh.at[pl.ds(s * RPS, RPS)])
    pltpu.sync_copy(ones_hbm, ones_v)
    plsc.subcore_barrier()

    def superblock(sb, carry):
        pltpu.sync_copy(dst_hbm.at[wid, sb], idx_d2)

        def body(j, c2):
            pltpu.async_copy(ones_v, deg_sh.at[idx_d2.at[j]], semd, add=True)

            @pl.when(j >= W_DEG)
            def _():
                pltpu.make_async_copy(ones_v, deg_sh.at[idx_d2.at[0]],
                                      semd).wait()

            return c2

        lax.fori_loop(0, IBD, body, 0)

        def drain(j, c2):
            pltpu.make_async_copy(ones_v, deg_sh.at[idx_d2.at[0]], semd).wait()
            return c2

        lax.fori_loop(0, W_DEG, drain, 0)
        return carry

    lax.fori_loop(0, NSBD, superblock, 0)
    plsc.subcore_barrier()
    pltpu.sync_copy(deg_sh.at[pl.ds(s * RPS, RPS)],
                    out_hbm.at[c, pl.ds(s * RPS, RPS)])


_sc_deg = pl.kernel(
    _sc_deg_body,
    out_type=jax.ShapeDtypeStruct((NC, N_PAD), jnp.float32),
    mesh=_MESH,
    scratch_types=[
        pltpu.VMEM_SHARED((N_PAD,), jnp.float32),
        pltpu.VMEM((IBD, KD), jnp.int32),
        pltpu.VMEM((KD,), jnp.float32),
        pltpu.SemaphoreType.DMA,
    ],
)


def _sc_scatter_body(hs_hbm, src_hbm, dst_hbm, zrows_hbm, out_hbm,
                     acc_sh, idx_s2, idx_d2, rows_v,
                     semg0, semg1, semg2, semg3,
                     sems0, sems1, sems2, sems3):
    semg = (semg0, semg1, semg2, semg3)
    sems = (sems0, sems1, sems2, sems3)
    c = lax.axis_index("c")
    s = lax.axis_index("s")
    wid = s * NC + c
    @pl.when(c == 0)
    def _():
        pltpu.sync_copy(hs_hbm.at[pl.ds(s * RPS, RPS)],
                        acc_sh.at[pl.ds(s * RPS, RPS)])

    @pl.when(c != 0)
    def _():
        pltpu.sync_copy(zrows_hbm, acc_sh.at[pl.ds(s * RPS, RPS)])

    plsc.subcore_barrier()

    def superblock(sb, carry):
        pltpu.sync_copy(src_hbm.at[wid, sb], idx_s2)
        pltpu.sync_copy(dst_hbm.at[wid, sb], idx_d2)
        for b in range(F_G):
            pltpu.async_copy(hs_hbm.at[idx_s2.at[b]], rows_v.at[b], semg[b])

        def inner(o, c2):
            for b in range(NB):
                j = o * NB + b
                pltpu.make_async_copy(hs_hbm.at[idx_s2.at[b]], rows_v.at[b],
                                      semg[b]).wait()
                pltpu.async_copy(rows_v.at[b], acc_sh.at[idx_d2.at[j]],
                                 sems[b], add=True)
                bq = (b + NB - Q_S) % NB

                @pl.when(j >= Q_S)
                def _():
                    pltpu.make_async_copy(rows_v.at[bq],
                                          acc_sh.at[idx_d2.at[j]],
                                          sems[bq]).wait()

                @pl.when(j + F_G < IB)
                def _():
                    pltpu.async_copy(hs_hbm.at[idx_s2.at[j + F_G]],
                                     rows_v.at[bq], semg[bq])

            return c2

        lax.fori_loop(0, IB // NB, inner, 0)
        for j in range(IB - Q_S, IB):
            pltpu.make_async_copy(rows_v.at[j % NB], acc_sh.at[idx_d2.at[0]],
                                  sems[j % NB]).wait()
        return carry

    lax.fori_loop(0, NSB, superblock, 0)
    plsc.subcore_barrier()
    pltpu.sync_copy(acc_sh.at[pl.ds(s * RPS, RPS)],
                    out_hbm.at[c, pl.ds(s * RPS, RPS)])


_sc_scatter = pl.kernel(
    _sc_scatter_body,
    out_type=jax.ShapeDtypeStruct((NC, N_PAD, H), jnp.float32),
    mesh=_MESH,
    scratch_types=[
        pltpu.VMEM_SHARED((N_PAD, H), jnp.float32),
        pltpu.VMEM((IB, KS), jnp.int32),
        pltpu.VMEM((IB, KS), jnp.int32),
        pltpu.VMEM((NB, KS, H), jnp.float32),
    ] + [pltpu.SemaphoreType.DMA] * (2 * NB),
)


def _hs_body(x_ref, w_ref, b_ref, d0_ref, d1_ref, hs_ref, dinv_ref):
    deg = d0_ref[...] + d1_ref[...] + 1.0
    dinv = lax.rsqrt(deg)
    h = jnp.dot(x_ref[...], w_ref[...], preferred_element_type=jnp.float32)
    hs_ref[...] = (h + b_ref[...]) * dinv
    dinv_ref[...] = dinv


_hs_call = pl.pallas_call(
    _hs_body,
    grid=(NBLK,),
    in_specs=[
        pl.BlockSpec((BLK, D), lambda i: (i, 0)),
        pl.BlockSpec((D, H), lambda i: (0, 0)),
        pl.BlockSpec((1, H), lambda i: (0, 0)),
        pl.BlockSpec((BLK, 1), lambda i: (i, 0)),
        pl.BlockSpec((BLK, 1), lambda i: (i, 0)),
    ],
    out_specs=[
        pl.BlockSpec((BLK, H), lambda i: (i, 0)),
        pl.BlockSpec((BLK, 1), lambda i: (i, 0)),
    ],
    out_shape=[
        jax.ShapeDtypeStruct((N_PAD, H), jnp.float32),
        jax.ShapeDtypeStruct((N, 1), jnp.float32),
    ],
)


def _final_body(a0_ref, a1_ref, dinv_ref, bi_ref, wfc_ref, bfc_ref,
                out_ref, sums_ref, cnts_ref):
    i = pl.program_id(0)

    @pl.when(i == 0)
    def _():
        sums_ref[...] = jnp.zeros_like(sums_ref)
        cnts_ref[...] = jnp.zeros_like(cnts_ref)

    node = jnp.maximum((a0_ref[...] + a1_ref[...]) * dinv_ref[...], 0.0)
    onehot = (bi_ref[...] == lax.broadcasted_iota(jnp.int32, (1, G), 1))
    onehot = onehot.astype(jnp.float32)
    sums_ref[...] += lax.dot_general(
        onehot, node, (((0,), (0,)), ((), ())),
        preferred_element_type=jnp.float32)
    cnts_ref[...] += lax.dot_general(
        onehot, jnp.ones_like(node), (((0,), (0,)), ((), ())),
        preferred_element_type=jnp.float32)

    @pl.when(i == NBLK - 1)
    def _():
        pooled = sums_ref[...] / jnp.maximum(cnts_ref[...], 1.0)
        logits = jnp.dot(pooled, wfc_ref[...],
                         preferred_element_type=jnp.float32) + bfc_ref[...]
        m = jnp.max(logits, axis=-1, keepdims=True)
        lse = m + jnp.log(jnp.sum(jnp.exp(logits - m), axis=-1, keepdims=True))
        out_ref[...] = logits - lse


_final_call = pl.pallas_call(
    _final_body,
    grid=(NBLK,),
    in_specs=[
        pl.BlockSpec((BLK, H), lambda i: (i, 0)),
        pl.BlockSpec((BLK, H), lambda i: (i, 0)),
        pl.BlockSpec((BLK, 1), lambda i: (i, 0)),
        pl.BlockSpec((BLK, 1), lambda i: (i, 0)),
        pl.BlockSpec((H, 2), lambda i: (0, 0)),
        pl.BlockSpec((1, 2), lambda i: (0, 0)),
    ],
    out_specs=pl.BlockSpec((G, 2), lambda i: (0, 0)),
    out_shape=jax.ShapeDtypeStruct((G, 2), jnp.float32),
    scratch_shapes=[
        pltpu.VMEM((G, H), jnp.float32),
        pltpu.VMEM((G, H), jnp.float32),
    ],
)


def kernel(x, edge_index, batch_idx, W1, b1, Wfc, bfc):
    dst_deg = edge_index[1].reshape(NW, NSBD, IBD, KD)
    pad_rows = N + jnp.arange(E_PAD - E, dtype=jnp.int32) % NPADR
    src_t = jnp.concatenate([edge_index[0], pad_rows]).reshape(
        NW, NSB, IB, KS)
    dst_t = jnp.concatenate([edge_index[1], pad_rows]).reshape(
        NW, NSB, IB, KS)
    bi = batch_idx.reshape(N, 1)
    ones_k = jnp.ones((KD,), jnp.float32)
    zeros_r = jnp.zeros((RPS,), jnp.float32)
    zrows = jnp.zeros((RPS, H), jnp.float32)

    deg_parts = _sc_deg(dst_deg, ones_k, zeros_r)
    d0 = deg_parts[0, :N].reshape(N, 1)
    d1 = deg_parts[1, :N].reshape(N, 1)
    hs, dinv = _hs_call(x, W1, b1.reshape(1, H), d0, d1)
    acc = _sc_scatter(hs, src_t, dst_t, zrows)
    return _final_call(acc[0], acc[1], dinv, bi, Wfc,
                       bfc.reshape(1, 2))
